# Initial kernel scaffold; baseline (speedup 1.0000x reference)
import jax
import jax.numpy as jnp
from jax.experimental import pallas as pl


def kernel(x, W_res, b_res, W_self0, W_nbr0, b0, W_self1, W_nbr1, b1, adj_value, edge_one, pool_val, edge_src, edge_dst, pool_idx):
    raise NotImplementedError("write your pallas kernel here")


# speedup vs baseline: 1.0004x; 1.0004x over previous
import functools

import jax
import jax.numpy as jnp
from jax.experimental import pallas as pl
from jax.experimental.pallas import tpu as pltpu

N = 10000
E = 320000
D = 128
NEXT_DIM = 2500


def _mm_bias_kernel(x_ref, w_ref, b_ref, o_ref):
    o_ref[...] = (
        jnp.dot(x_ref[...], w_ref[...], preferred_element_type=jnp.float32)
        + b_ref[...]
    )


def _mm_bias(x2, W, b, block=1000):
    n = x2.shape[0]
    dout = W.shape[1]
    return pl.pallas_call(
        _mm_bias_kernel,
        grid=(n // block,),
        in_specs=[
            pl.BlockSpec((block, x2.shape[1]), lambda i: (i, 0)),
            pl.BlockSpec((x2.shape[1], dout), lambda i: (0, 0)),
            pl.BlockSpec((1, dout), lambda i: (0, 0)),
        ],
        out_specs=pl.BlockSpec((block, dout), lambda i: (i, 0)),
        out_shape=jax.ShapeDtypeStruct((n, dout), jnp.float32),
    )(x2, W, b.reshape(1, dout))


def kernel(x, W_res, b_res, W_self0, W_nbr0, b0, W_self1, W_nbr1, b1,
           adj_value, edge_one, pool_val, edge_src, edge_dst, pool_idx):
    x2 = x[0]

    def ecc(h, W_self, W_nbr, b):
        msg = h[edge_dst] * adj_value[:, None]
        agg = jnp.zeros((N, D), jnp.float32).at[edge_src].add(msg)
        deg = jnp.zeros((N,), jnp.float32).at[edge_src].add(edge_one)
        deg = jnp.clip(deg, 1e-6, None)
        agg = agg / deg[:, None]
        return _mm_bias(h, W_self, b) + agg @ W_nbr

    def pool(h):
        rows, cols = pool_idx[0], pool_idx[1]
        gathered = h[cols] * pool_val[:, None]
        return jnp.zeros((NEXT_DIM, D), jnp.float32).at[rows].add(gathered)

    residual = _mm_bias(x2, W_res.T, b_res)
    out = ecc(x2, W_self0, W_nbr0, b0)
    out = jax.nn.leaky_relu(out, 0.2)
    out = ecc(out, W_self1, W_nbr1, b1)
    return (pool(out) + pool(residual))[None]


# speedup vs baseline: 4.0929x; 4.0914x over previous
import functools

import jax
import jax.numpy as jnp
from jax import lax
from jax.experimental import pallas as pl
from jax.experimental.pallas import tpu as pltpu
from jax.experimental.pallas import tpu_sc as plsc

N = 10000
E = 320000
D = 128
H = D // 2
P = 2500

NT = 16
EPT = 20480
CH = 128
NCH = EPT // CH
AGG_ROWS = 10240
EPAD = NT * EPT - E

PW = 16
PPW = N // PW
PCH = 125
PNCH = PPW // PCH
PACC_ROWS = 2560


def _zero_fill(ref, nrows, ncols):
    z = jnp.zeros((16,), jnp.float32)

    def body(k, _):
        for j in range(ncols // 16):
            ref[k, pl.ds(j * 16, 16)] = z
        return 0

    lax.fori_loop(0, nrows, body, 0, unroll=2)


def _make_spmm(with_deg):
    mesh = plsc.VectorSubcoreMesh(core_axis_name="c", subcore_axis_name="s")
    out_type = [jax.ShapeDtypeStruct((N, H), jnp.float32),
                jax.ShapeDtypeStruct((N, H), jnp.float32)]
    if with_deg:
        out_type.append(jax.ShapeDtypeStruct((N, 16), jnp.float32))
    scratch = [
        pltpu.VMEM((NCH, CH), jnp.int32),
        pltpu.VMEM((NCH, CH), jnp.int32),
        pltpu.VMEM((2, CH, 16), jnp.float32),
        pltpu.VMEM((2, CH, H), jnp.float32),
        pltpu.VMEM((CH, 16), jnp.float32),
        pltpu.VMEM_SHARED((AGG_ROWS, H), jnp.float32),
        pltpu.VMEM_SHARED((AGG_ROWS, 16), jnp.float32),
        pltpu.SemaphoreType.DMA,
        pltpu.SemaphoreType.DMA,
    ]

    def body(x_lo, x_hi, dst3, src3, adj16, *rest):
        if with_deg:
            agg_lo, agg_hi, deg_o = rest[:3]
            rest = rest[3:]
        else:
            agg_lo, agg_hi = rest[:2]
            rest = rest[2:]
        (dst_v, src_v, adj_v, rows_v, ones_v, agg_sh, deg_sh,
         sem0, sem1) = rest
        sems = (sem0, sem1)
        xs = (x_lo, x_hi)
        outs = (agg_lo, agg_hi)

        c = lax.axis_index("c")
        s = lax.axis_index("s")

        pltpu.sync_copy(dst3.at[s], dst_v)
        pltpu.sync_copy(src3.at[s], src_v)

        _zero_fill(rows_v.at[0], CH, H)
        rows0 = AGG_ROWS // NT
        for k in range(rows0 // CH):
            pltpu.sync_copy(rows_v.at[0],
                            agg_sh.at[pl.ds(s * rows0 + k * CH, CH)])
        do_deg = with_deg

        if do_deg:
            @pl.when(c == 0)
            def _():
                _zero_fill(ones_v, CH, 16)
                for k in range(rows0 // CH):
                    pltpu.sync_copy(ones_v,
                                    deg_sh.at[pl.ds(s * rows0 + k * CH, CH)])
                one = jnp.ones((16,), jnp.float32)

                def ones_body(k, _):
                    ones_v[k, pl.ds(0, 16)] = one
                    return 0

                lax.fori_loop(0, CH, ones_body, 0, unroll=4)

        plsc.subcore_barrier()

        for cc in range(2):
            @pl.when(c == cc)
            def _(cc=cc):
                x_hbm = xs[cc]

                def start_gather(ch, b):
                    pltpu.async_copy(x_hbm.at[dst_v.at[ch]], rows_v.at[b],
                                     sems[b])
                    pltpu.async_copy(adj16.at[s, ch], adj_v.at[b], sems[b])

                def wait_gather(ch, b):
                    pltpu.make_async_copy(x_hbm.at[dst_v.at[ch]],
                                          rows_v.at[b], sems[b]).wait()
                    pltpu.make_async_copy(adj16.at[s, ch], adj_v.at[b],
                                          sems[b]).wait()

                start_gather(0, 0)
                start_gather(1, 1)

                def chunk_body(it, _):
                    for b in range(2):
                        ch = it * 2 + b
                        wait_gather(ch, b)

                        def scale_body(k, _):
                            sc = adj_v[b, k, pl.ds(0, 16)]
                            for j in range(H // 16):
                                sl = pl.ds(j * 16, 16)
                                rows_v[b, k, sl] = rows_v[b, k, sl] * sc
                            return 0

                        lax.fori_loop(0, CH, scale_body, 0, unroll=2)

                        pltpu.sync_copy(rows_v.at[b],
                                        agg_sh.at[src_v.at[ch]], add=True)
                        if do_deg and cc == 0:
                            pltpu.sync_copy(ones_v,
                                            deg_sh.at[src_v.at[ch]],
                                            add=True)

                        @pl.when(it < (NCH // 2) - 1)
                        def _():
                            start_gather(ch + 2, b)
                    return 0

                lax.fori_loop(0, NCH // 2, chunk_body, 0)

        plsc.subcore_barrier()

        rows0 = AGG_ROWS // NT
        for cc in range(2):
            @pl.when(c == cc)
            def _(cc=cc):
                @pl.when(s < NT - 1)
                def _():
                    pltpu.sync_copy(agg_sh.at[pl.ds(s * rows0, rows0)],
                                    outs[cc].at[pl.ds(s * rows0, rows0)])
                    if with_deg and cc == 0:
                        pltpu.sync_copy(deg_sh.at[pl.ds(s * rows0, rows0)],
                                        deg_o.at[pl.ds(s * rows0, rows0)])

                @pl.when(s == NT - 1)
                def _():
                    last = N - (NT - 1) * rows0
                    pltpu.sync_copy(
                        agg_sh.at[pl.ds((NT - 1) * rows0, last)],
                        outs[cc].at[pl.ds((NT - 1) * rows0, last)])
                    if with_deg and cc == 0:
                        pltpu.sync_copy(
                            deg_sh.at[pl.ds((NT - 1) * rows0, last)],
                            deg_o.at[pl.ds((NT - 1) * rows0, last)])

    return pl.kernel(body, out_type=tuple(out_type), mesh=mesh,
                     scratch_types=scratch,
                     compiler_params=pltpu.CompilerParams(
                         use_tc_tiling_on_sc=False),
                     name="spmm_deg" if with_deg else "spmm")


_spmm_deg = _make_spmm(True)
_spmm = _make_spmm(False)


def _pool_body(z_hbm, cols3, rows3, val16, out, cols_v, rows_v, val_v,
               zrows_v, pacc_sh, sem0):
    c = lax.axis_index("c")
    s = lax.axis_index("s")

    @pl.when(c == 0)
    def _():
        pltpu.sync_copy(cols3.at[s], cols_v)
        pltpu.sync_copy(rows3.at[s], rows_v)

        _zero_fill(zrows_v, PCH, D)
        rows0 = PACC_ROWS // 16
        pltpu.sync_copy(zrows_v.at[pl.ds(0, 80)],
                        pacc_sh.at[pl.ds(s * rows0, 80)])
        pltpu.sync_copy(zrows_v.at[pl.ds(0, 80)],
                        pacc_sh.at[pl.ds(s * rows0 + 80, 80)])

        plsc.subcore_barrier()

        def chunk(ch, _):
            pltpu.async_copy(z_hbm.at[cols_v.at[ch]], zrows_v, sem0)
            pltpu.sync_copy(val16.at[s, ch], val_v)
            pltpu.make_async_copy(z_hbm.at[cols_v.at[ch]], zrows_v,
                                  sem0).wait()

            def scale_body(k, _):
                sc = val_v[k, pl.ds(0, 16)]
                for j in range(D // 16):
                    sl = pl.ds(j * 16, 16)
                    zrows_v[k, sl] = zrows_v[k, sl] * sc
                return 0

            lax.fori_loop(0, PCH, scale_body, 0, unroll=2)
            pltpu.sync_copy(zrows_v, pacc_sh.at[rows_v.at[ch]], add=True)
            return 0

        lax.fori_loop(0, PNCH, chunk, 0)

        plsc.subcore_barrier()

        @pl.when(s < 9)
        def _():
            pltpu.sync_copy(pacc_sh.at[pl.ds(s * 256, 256)],
                            out.at[pl.ds(s * 256, 256)])

        @pl.when(s == 9)
        def _():
            pltpu.sync_copy(pacc_sh.at[pl.ds(9 * 256, P - 9 * 256)],
                            out.at[pl.ds(9 * 256, P - 9 * 256)])


_pool = pl.kernel(
    _pool_body,
    out_type=jax.ShapeDtypeStruct((P, D), jnp.float32),
    mesh=plsc.VectorSubcoreMesh(core_axis_name="c", subcore_axis_name="s"),
    scratch_types=[
        pltpu.VMEM((PNCH, PCH), jnp.int32),
        pltpu.VMEM((PNCH, PCH), jnp.int32),
        pltpu.VMEM((PCH, 16), jnp.float32),
        pltpu.VMEM((PCH, D), jnp.float32),
        pltpu.VMEM_SHARED((PACC_ROWS, D), jnp.float32),
        pltpu.SemaphoreType.DMA,
    ],
    compiler_params=pltpu.CompilerParams(use_tc_tiling_on_sc=False),
    name="pool")


def _mm_bias_kernel(x_ref, w_ref, b_ref, o_ref):
    o_ref[...] = (
        jnp.dot(x_ref[...], w_ref[...], preferred_element_type=jnp.float32)
        + b_ref[...]
    )


def _mm_bias(x2, W, b, block=2000):
    n, din = x2.shape
    dout = W.shape[1]
    return pl.pallas_call(
        _mm_bias_kernel,
        grid=(n // block,),
        in_specs=[
            pl.BlockSpec((block, din), lambda i: (i, 0)),
            pl.BlockSpec((din, dout), lambda i: (0, 0)),
            pl.BlockSpec((1, dout), lambda i: (0, 0)),
        ],
        out_specs=pl.BlockSpec((block, dout), lambda i: (i, 0)),
        out_shape=jax.ShapeDtypeStruct((n, dout), jnp.float32),
    )(x2, W, b.reshape(1, dout))


def _mix_kernel(leaky, s_ref, al_ref, ah_ref, d_ref, w_ref, ol_ref, oh_ref):
    agg = jnp.concatenate([al_ref[...], ah_ref[...]], axis=1)
    inv = 1.0 / jnp.maximum(d_ref[:, 0:1], 1e-6)
    v = s_ref[...] + jnp.dot(agg * inv, w_ref[...],
                             preferred_element_type=jnp.float32)
    if leaky:
        v = jnp.where(v >= 0, v, 0.2 * v)
    ol_ref[...] = v[:, :H]
    oh_ref[...] = v[:, H:]


def _mix(selfp, agg_lo, agg_hi, deg, W_nbr, leaky, block=2000):
    n = selfp.shape[0]
    return pl.pallas_call(
        functools.partial(_mix_kernel, leaky),
        grid=(n // block,),
        in_specs=[
            pl.BlockSpec((block, D), lambda i: (i, 0)),
            pl.BlockSpec((block, H), lambda i: (i, 0)),
            pl.BlockSpec((block, H), lambda i: (i, 0)),
            pl.BlockSpec((block, 16), lambda i: (i, 0)),
            pl.BlockSpec((D, D), lambda i: (0, 0)),
        ],
        out_specs=[
            pl.BlockSpec((block, H), lambda i: (i, 0)),
            pl.BlockSpec((block, H), lambda i: (i, 0)),
        ],
        out_shape=[
            jax.ShapeDtypeStruct((n, H), jnp.float32),
            jax.ShapeDtypeStruct((n, H), jnp.float32),
        ],
    )(selfp, agg_lo, agg_hi, deg, W_nbr)


def _mix_full_kernel(s_ref, al_ref, ah_ref, d_ref, w_ref, o_ref):
    agg = jnp.concatenate([al_ref[...], ah_ref[...]], axis=1)
    inv = 1.0 / jnp.maximum(d_ref[:, 0:1], 1e-6)
    o_ref[...] = s_ref[...] + jnp.dot(agg * inv, w_ref[...],
                                      preferred_element_type=jnp.float32)


def _mix_full(selfp, agg_lo, agg_hi, deg, W_nbr, block=2000):
    n = selfp.shape[0]
    return pl.pallas_call(
        _mix_full_kernel,
        grid=(n // block,),
        in_specs=[
            pl.BlockSpec((block, D), lambda i: (i, 0)),
            pl.BlockSpec((block, H), lambda i: (i, 0)),
            pl.BlockSpec((block, H), lambda i: (i, 0)),
            pl.BlockSpec((block, 16), lambda i: (i, 0)),
            pl.BlockSpec((D, D), lambda i: (0, 0)),
        ],
        out_specs=pl.BlockSpec((block, D), lambda i: (i, 0)),
        out_shape=jax.ShapeDtypeStruct((n, D), jnp.float32),
    )(selfp, agg_lo, agg_hi, deg, W_nbr)


def _mm2_add_kernel(hl_ref, hh_ref, w_ref, r_ref, b_ref, o_ref):
    h = jnp.concatenate([hl_ref[...], hh_ref[...]], axis=1)
    o_ref[...] = (
        jnp.dot(h, w_ref[...], preferred_element_type=jnp.float32)
        + r_ref[...] + b_ref[...]
    )


def _mm2_add(h_lo, h_hi, W, r, b, block=2000):
    n = h_lo.shape[0]
    return pl.pallas_call(
        _mm2_add_kernel,
        grid=(n // block,),
        in_specs=[
            pl.BlockSpec((block, H), lambda i: (i, 0)),
            pl.BlockSpec((block, H), lambda i: (i, 0)),
            pl.BlockSpec((D, D), lambda i: (0, 0)),
            pl.BlockSpec((block, D), lambda i: (i, 0)),
            pl.BlockSpec((1, D), lambda i: (0, 0)),
        ],
        out_specs=pl.BlockSpec((block, D), lambda i: (i, 0)),
        out_shape=jax.ShapeDtypeStruct((n, D), jnp.float32),
    )(h_lo, h_hi, W, r, b.reshape(1, D))


def kernel(x, W_res, b_res, W_self0, W_nbr0, b0, W_self1, W_nbr1, b1,
           adj_value, edge_one, pool_val, edge_src, edge_dst, pool_idx):
    x2 = x[0]
    x_lo = x2[:, :H]
    x_hi = x2[:, H:]

    dst3 = jnp.concatenate(
        [edge_dst, jnp.zeros((EPAD,), jnp.int32)]).reshape(NT, NCH, CH)
    src3 = jnp.concatenate(
        [edge_src, jnp.full((EPAD,), N, jnp.int32)]).reshape(NT, NCH, CH)
    adj16 = jnp.broadcast_to(
        jnp.concatenate([adj_value, jnp.zeros((EPAD,), jnp.float32)])[:, None],
        (NT * EPT, 16)).reshape(NT, NCH, CH, 16)

    cols3 = pool_idx[1].reshape(PW, PNCH, PCH)
    rows3 = pool_idx[0].reshape(PW, PNCH, PCH)
    val16 = jnp.broadcast_to(
        pool_val[:, None], (N, 16)).reshape(PW, PNCH, PCH, 16)

    agg0l, agg0h, deg = _spmm_deg(x_lo, x_hi, dst3, src3, adj16)
    Wcat = jnp.concatenate([W_res.T, W_self0], axis=1)
    bcat = jnp.concatenate([b_res, b0])
    Y = _mm_bias(x2, Wcat, bcat)
    residual = Y[:, :D]
    self0 = Y[:, D:]

    h_lo, h_hi = _mix(self0, agg0l, agg0h, deg, W_nbr0, leaky=True)

    agg1l, agg1h = _spmm(h_lo, h_hi, dst3, src3, adj16)
    selfres = _mm2_add(h_lo, h_hi, W_self1, residual, b1)

    z = _mix_full(selfres, agg1l, agg1h, deg, W_nbr1)

    return _pool(z, cols3, rows3, val16)[None]
